# Initial kernel scaffold; baseline (speedup 1.0000x reference)
import jax
import jax.numpy as jnp
from jax.experimental import pallas as pl


def kernel(edgeIdOfPath, pathSegmentFeat, pathNum, edge_src, edge_dst, orderInfo, segFeature, emb_table, W_inner, b_inner, W_r1, b_r1, W_r2, b_r2, W_seg, W_g1, b_g1, W_g2, b_g2):
    raise NotImplementedError("write your pallas kernel here")


# speedup vs baseline: 1.5048x; 1.5048x over previous
import jax
import jax.numpy as jnp
from jax.experimental import pallas as pl
from jax.experimental.pallas import tpu as pltpu

P = 10000
L = 32
EDGE_NUM = 10000
HORIZON = 4

BP = 1000
BS = 1000
PK = 10240
BK = 2048


def _path_kernel(ids_ref, e_ref, f0_ref, f1_ref, wi_e_ref, wi_f0_ref,
                 wi_f1_ref, bi_ref, wr1_ref, br1_ref, wr2_ref, br2_ref,
                 pe_ref, pf_ref):
    ids = ids_ref[...]
    mask = (ids != 0).astype(jnp.float32)
    lengths = jnp.sum(mask, axis=1, keepdims=True)
    inv = 1.0 / jnp.maximum(lengths, 1.0)
    e = e_ref[...]
    pooled_e = e[:, 0:48]
    for l in range(1, L):
        pooled_e = pooled_e + e[:, l * 48:(l + 1) * 48]
    pooled_e = pooled_e * inv
    pf0 = jnp.sum(f0_ref[...] * mask, axis=1, keepdims=True) * inv
    pf1 = jnp.sum(f1_ref[...] * mask, axis=1, keepdims=True) * inv
    z = (jnp.dot(pooled_e, wi_e_ref[...], preferred_element_type=jnp.float32)
         + pf0 * wi_f0_ref[...] + pf1 * wi_f1_ref[...] + bi_ref[...])
    pe = jnp.tanh(z)
    pe_ref[...] = pe
    h = jnp.maximum(
        jnp.dot(pe, wr1_ref[...], preferred_element_type=jnp.float32)
        + br1_ref[...], 0.0)
    pf_ref[...] = (jnp.dot(h, wr2_ref[...], preferred_element_type=jnp.float32)
                   + br2_ref[...])


def _seg_kernel(a_ref, pe_ref, s0_ref, s1_ref, wseg0_ref, wseg1_ref,
                wg1_ref, bg1_ref, wg2_ref, bg2_ref, out_ref, acc_ref):
    k = pl.program_id(1)
    nk = pl.num_programs(1)

    @pl.when(k == 0)
    def _init():
        acc_ref[...] = jnp.zeros_like(acc_ref)

    acc_ref[...] += jnp.dot(a_ref[...], pe_ref[...],
                            preferred_element_type=jnp.float32)

    @pl.when(k == nk - 1)
    def _finish():
        seg_h = (acc_ref[...] + s0_ref[...] * wseg0_ref[...]
                 + s1_ref[...] * wseg1_ref[...])
        g = jnp.maximum(
            jnp.dot(seg_h, wg1_ref[...], preferred_element_type=jnp.float32)
            + bg1_ref[...], 0.0)
        out_ref[...] = (jnp.dot(g, wg2_ref[...],
                                preferred_element_type=jnp.float32)
                        + bg2_ref[...])


def kernel(edgeIdOfPath, pathSegmentFeat, pathNum, edge_src, edge_dst,
           orderInfo, segFeature, emb_table, W_inner, b_inner, W_r1, b_r1,
           W_r2, b_r2, W_seg, W_g1, b_g1, W_g2, b_g2):
    ids = edgeIdOfPath.astype(jnp.int32)
    emb0 = emb_table.at[0].set(0.0)
    e_flat = jnp.take(emb0, ids, axis=0).reshape(P, L * 48)
    f0 = pathSegmentFeat[:, :, 0]
    f1 = pathSegmentFeat[:, :, 1]

    wi_e = W_inner[:48]
    wi_f0 = W_inner[48:49]
    wi_f1 = W_inner[49:50]
    bi = b_inner[None, :]
    br1 = b_r1[None, :]
    br2 = b_r2[None, :]

    pathEmbedding, predFlow = pl.pallas_call(
        _path_kernel,
        grid=(P // BP,),
        in_specs=[
            pl.BlockSpec((BP, L), lambda i: (i, 0)),
            pl.BlockSpec((BP, L * 48), lambda i: (i, 0)),
            pl.BlockSpec((BP, L), lambda i: (i, 0)),
            pl.BlockSpec((BP, L), lambda i: (i, 0)),
            pl.BlockSpec((48, 640), lambda i: (0, 0)),
            pl.BlockSpec((1, 640), lambda i: (0, 0)),
            pl.BlockSpec((1, 640), lambda i: (0, 0)),
            pl.BlockSpec((1, 640), lambda i: (0, 0)),
            pl.BlockSpec((640, 1024), lambda i: (0, 0)),
            pl.BlockSpec((1, 1024), lambda i: (0, 0)),
            pl.BlockSpec((1024, 1), lambda i: (0, 0)),
            pl.BlockSpec((1, 1), lambda i: (0, 0)),
        ],
        out_specs=[
            pl.BlockSpec((BP, 640), lambda i: (i, 0)),
            pl.BlockSpec((BP, 1), lambda i: (i, 0)),
        ],
        out_shape=[
            jax.ShapeDtypeStruct((P, 640), jnp.float32),
            jax.ShapeDtypeStruct((P, 1), jnp.float32),
        ],
    )(ids, e_flat, f0, f1, wi_e, wi_f0, wi_f1, bi, W_r1, br1, W_r2, br2)

    A = jnp.zeros((EDGE_NUM, PK), jnp.float32).at[edge_dst, edge_src].add(
        orderInfo[:, 0])
    pe_pad = jnp.pad(pathEmbedding, ((0, PK - P), (0, 0)))

    s0 = segFeature[:, 0:1]
    s1 = segFeature[:, 1:2]
    wseg0 = W_seg[0:1]
    wseg1 = W_seg[1:2]
    bg1 = b_g1[None, :]
    bg2 = b_g2[None, :]

    out = pl.pallas_call(
        _seg_kernel,
        grid=(EDGE_NUM // BS, PK // BK),
        in_specs=[
            pl.BlockSpec((BS, BK), lambda s, k: (s, k)),
            pl.BlockSpec((BK, 640), lambda s, k: (k, 0)),
            pl.BlockSpec((BS, 1), lambda s, k: (s, 0)),
            pl.BlockSpec((BS, 1), lambda s, k: (s, 0)),
            pl.BlockSpec((1, 640), lambda s, k: (0, 0)),
            pl.BlockSpec((1, 640), lambda s, k: (0, 0)),
            pl.BlockSpec((640, 1200), lambda s, k: (0, 0)),
            pl.BlockSpec((1, 1200), lambda s, k: (0, 0)),
            pl.BlockSpec((1200, HORIZON), lambda s, k: (0, 0)),
            pl.BlockSpec((1, HORIZON), lambda s, k: (0, 0)),
        ],
        out_specs=pl.BlockSpec((BS, HORIZON), lambda s, k: (s, 0)),
        out_shape=jax.ShapeDtypeStruct((EDGE_NUM, HORIZON), jnp.float32),
        scratch_shapes=[pltpu.VMEM((BS, 640), jnp.float32)],
    )(A, pe_pad, s0, s1, wseg0, wseg1, W_g1, bg1, W_g2, bg2)

    return (out, predFlow, pathNum)
